# baseline (speedup 1.0000x reference)
import functools

import jax
import jax.numpy as jnp
from jax import lax
from jax.experimental import pallas as pl
from jax.experimental.pallas import tpu as pltpu
from jax.experimental.pallas import tpu_sc as plsc

TOTAL = 16384
VOCAB = 4096
NAST = 16
MAXV = 2048
NSUB = 16
CHUNK = TOTAL // NSUB
L = 16
SEGR = 64
SEGS = CHUNK // SEGR
NBUF = 8


def _body(tbl, ids, wts, ridx, rmask, out_ast, out_ppl, out_m,
          ids_v, w_v, flat_v, rows_v, wll_v, macc_v, ridx_v, rmask_v,
          wll_full, metrics_l, stage_v,
          wll_sh,
          sem_in, gsems, sem_r):
    sid = lax.axis_index("s")
    base = sid * CHUNK

    cp_ids = pltpu.async_copy(ids.at[pl.ds(base, CHUNK)], ids_v, sem_in)
    cp_w = pltpu.async_copy(wts.at[pl.ds(base, CHUNK)], w_v, sem_in)
    cp_ri = pltpu.async_copy(ridx.at[sid], ridx_v, sem_r)
    cp_rm = pltpu.async_copy(rmask.at[sid], rmask_v, sem_r)
    cp_ids.wait()
    cp_w.wait()

    iota = lax.iota(jnp.int32, L)
    for j in range(CHUNK // L):
        v = ids_v[pl.ds(j * L, L)]
        rows = (base + j * L) + iota
        flat_v[j // (SEGR // L), pl.ds((j % (SEGR // L)) * L, L)] = (
            rows * (VOCAB // 128) + lax.shift_right_logical(v, 7))

    zero = jnp.zeros((L,), jnp.float32)
    one = jnp.ones((L,), jnp.float32)
    sr = zero
    nr = zero
    su = zero
    nu = zero

    def _extract(s, b):
        nonlocal sr, nr, su, nu
        buf = rows_v.at[b]
        for q in range(SEGR // L):
            j = s * (SEGR // L) + q
            v = ids_v[pl.ds(j * L, L)]
            ll = plsc.load_gather(buf, [q * L + iota, v & 127])
            w = w_v[pl.ds(j * L, L)]
            rm = jnp.where(w == 1.0, one, zero)
            lr = ll * rm
            sr = sr + lr
            nr = nr + rm
            su = su + (ll - lr)
            nu = nu + (one - rm)
            wll_v[pl.ds(j * L, L)] = ll * w

    cps = [None] * NBUF
    for s in range(NBUF):
        cps[s] = pltpu.async_copy(tbl.at[flat_v.at[s]], rows_v.at[s],
                                  gsems.at[s])
    for s in range(SEGS):
        b = s % NBUF
        cps[b].wait()
        _extract(s, b)
        if s + NBUF < SEGS:
            cps[b] = pltpu.async_copy(tbl.at[flat_v.at[s + NBUF]],
                                      rows_v.at[b], gsems.at[b])
    macc_v[0, :] = sr
    macc_v[1, :] = nr
    macc_v[2, :] = su
    macc_v[3, :] = nu
    pltpu.sync_copy(wll_v, wll_sh.at[pl.ds(base, CHUNK)])
    pltpu.sync_copy(macc_v, out_m.at[sid])
    plsc.subcore_barrier()

    @pl.when(sid == 0)
    def _ppl():
        pltpu.sync_copy(out_m, metrics_l)
        sr_t = zero
        nr_t = zero
        su_t = zero
        nu_t = zero
        for t in range(NSUB):
            sr_t = sr_t + metrics_l[t, 0, :]
            nr_t = nr_t + metrics_l[t, 1, :]
            su_t = su_t + metrics_l[t, 2, :]
            nu_t = nu_t + metrics_l[t, 3, :]
        ssr = jnp.full((L,), jnp.sum(sr_t))
        snr = jnp.full((L,), jnp.sum(nr_t))
        ssu = jnp.full((L,), jnp.sum(su_t))
        snu = jnp.full((L,), jnp.sum(nu_t))
        rv = jnp.exp(-(ssr / snr))
        uv = jnp.exp(-(ssu / snu))
        stage_v[...] = jnp.where(iota == 0, rv, jnp.where(iota == 1, uv, zero))
        pltpu.sync_copy(stage_v, out_ppl)

    pltpu.sync_copy(wll_sh, wll_full)
    cp_ri.wait()
    cp_rm.wait()
    acc = zero
    for k in range(MAXV // L):
        idx = ridx_v[pl.ds(k * L, L)]
        vals = plsc.load_gather(wll_full, [idx])
        m = rmask_v[pl.ds(k * L, L)]
        acc = acc + vals * m
    stage_v[...] = jnp.full((L,), jnp.sum(acc))
    pltpu.sync_copy(stage_v, out_ast.at[sid])


_sc_call = functools.partial(
    pl.kernel,
    out_type=[
        jax.ShapeDtypeStruct((NAST, L), jnp.float32),
        jax.ShapeDtypeStruct((L,), jnp.float32),
        jax.ShapeDtypeStruct((NSUB, 4, L), jnp.float32),
    ],
    mesh=plsc.VectorSubcoreMesh(core_axis_name="c", subcore_axis_name="s",
                                num_cores=1),
    compiler_params=pltpu.CompilerParams(needs_layout_passes=False),
    scratch_types=[
        pltpu.VMEM((CHUNK,), jnp.int32),
        pltpu.VMEM((CHUNK,), jnp.float32),
        pltpu.VMEM((SEGS, SEGR), jnp.int32),
        pltpu.VMEM((NBUF, SEGR, 128), jnp.float32),
        pltpu.VMEM((CHUNK,), jnp.float32),
        pltpu.VMEM((4, L), jnp.float32),
        pltpu.VMEM((MAXV,), jnp.int32),
        pltpu.VMEM((MAXV,), jnp.float32),
        pltpu.VMEM((TOTAL,), jnp.float32),
        pltpu.VMEM((NSUB, 4, L), jnp.float32),
        pltpu.VMEM((L,), jnp.float32),
        pltpu.VMEM_SHARED((TOTAL,), jnp.float32),
        pltpu.SemaphoreType.DMA,
        pltpu.SemaphoreType.DMA((NBUF,)),
        pltpu.SemaphoreType.DMA,
    ],
)(_body)


def kernel(var_name_log_probs, variable_tgt_name_id, variable_tgt_name_weight,
           restoration_indices, restoration_mask):
    tbl_rows = var_name_log_probs.reshape(TOTAL * VOCAB // 128, 128)
    out_ast, out_ppl, _ = _sc_call(tbl_rows, variable_tgt_name_id,
                                   variable_tgt_name_weight,
                                   restoration_indices, restoration_mask)
    return (out_ast[:, 0], out_ppl[0], out_ppl[1])


# speedup vs baseline: 2.4041x; 2.4041x over previous
import functools

import jax
import jax.numpy as jnp
from jax import lax
from jax.experimental import pallas as pl
from jax.experimental.pallas import tpu as pltpu
from jax.experimental.pallas import tpu_sc as plsc

TOTAL = 16384
VOCAB = 4096
NAST = 16
MAXV = 2048
NSUB = 16
NW = 32
L = 16
VARS_W = TOTAL // NW
CROWS = 8
NCHUNK = VARS_W // CROWS


def _scan_body(tbl, ids, wts, out_wll, out_m,
               ids_v, w_v, wll_v, macc_v, bufs, sem_in, gsems):
    sid = lax.axis_index("s")
    cid = lax.axis_index("c")
    wid = sid * 2 + cid
    base = wid * VARS_W

    cp_ids = pltpu.async_copy(ids.at[pl.ds(base, VARS_W)], ids_v, sem_in)
    cp_w = pltpu.async_copy(wts.at[pl.ds(base, VARS_W)], w_v, sem_in)

    cps = [None, None]
    for b in range(2):
        cps[b] = pltpu.async_copy(tbl.at[pl.ds(base + b * CROWS, CROWS)],
                                  bufs.at[b], gsems.at[b])
    cp_ids.wait()
    cp_w.wait()

    iota = lax.iota(jnp.int32, L)
    low = iota < 8
    zero = jnp.zeros((L,), jnp.float32)
    one = jnp.ones((L,), jnp.float32)
    sr = zero
    nr = zero
    su = zero
    nu = zero
    pending = zero
    for c in range(NCHUNK):
        b = c & 1
        p = c // 2
        cps[b].wait()
        v = ids_v[pl.ds(p * L, L)]
        vals = plsc.load_gather(bufs.at[b], [iota & 7, v])
        if c % 2 == 0:
            pending = vals
        else:
            ll = jnp.where(low, pending, vals)
            w = w_v[pl.ds(p * L, L)]
            rm = jnp.where(w == 1.0, one, zero)
            lr = ll * rm
            sr = sr + lr
            nr = nr + rm
            su = su + (ll - lr)
            nu = nu + (one - rm)
            wll_v[pl.ds(p * L, L)] = ll * w
        if c + 2 < NCHUNK:
            cps[b] = pltpu.async_copy(
                tbl.at[pl.ds(base + (c + 2) * CROWS, CROWS)],
                bufs.at[b], gsems.at[b])
    macc_v[0, :] = sr
    macc_v[1, :] = nr
    macc_v[2, :] = su
    macc_v[3, :] = nu
    pltpu.sync_copy(wll_v, out_wll.at[pl.ds(base, VARS_W)])
    pltpu.sync_copy(macc_v, out_m.at[wid])


_scan_call = functools.partial(
    pl.kernel,
    out_type=[
        jax.ShapeDtypeStruct((TOTAL,), jnp.float32),
        jax.ShapeDtypeStruct((NW, 4, L), jnp.float32),
    ],
    mesh=plsc.VectorSubcoreMesh(core_axis_name="c", subcore_axis_name="s"),
    compiler_params=pltpu.CompilerParams(needs_layout_passes=False),
    scratch_types=[
        pltpu.VMEM((VARS_W,), jnp.int32),
        pltpu.VMEM((VARS_W,), jnp.float32),
        pltpu.VMEM((VARS_W,), jnp.float32),
        pltpu.VMEM((4, L), jnp.float32),
        pltpu.VMEM((2, CROWS, VOCAB), jnp.float32),
        pltpu.SemaphoreType.DMA,
        pltpu.SemaphoreType.DMA((2,)),
    ],
)(_scan_body)


def _ast_body(wll, m32, ridx, rmask, out_ast, out_ppl,
              wll_full, ridx_v, rmask_v, metrics_l, stage_v, sem):
    sid = lax.axis_index("s")
    iota = lax.iota(jnp.int32, L)
    zero = jnp.zeros((L,), jnp.float32)

    cp_w = pltpu.async_copy(wll, wll_full, sem)
    cp_ri = pltpu.async_copy(ridx.at[sid], ridx_v, sem)
    cp_rm = pltpu.async_copy(rmask.at[sid], rmask_v, sem)

    @pl.when(sid == 0)
    def _ppl():
        pltpu.sync_copy(m32, metrics_l)
        sr_t = zero
        nr_t = zero
        su_t = zero
        nu_t = zero
        for t in range(NW):
            sr_t = sr_t + metrics_l[t, 0, :]
            nr_t = nr_t + metrics_l[t, 1, :]
            su_t = su_t + metrics_l[t, 2, :]
            nu_t = nu_t + metrics_l[t, 3, :]
        ssr = jnp.full((L,), jnp.sum(sr_t))
        snr = jnp.full((L,), jnp.sum(nr_t))
        ssu = jnp.full((L,), jnp.sum(su_t))
        snu = jnp.full((L,), jnp.sum(nu_t))
        rv = jnp.exp(-(ssr / snr))
        uv = jnp.exp(-(ssu / snu))
        stage_v[...] = jnp.where(iota == 0, rv, jnp.where(iota == 1, uv, zero))
        pltpu.sync_copy(stage_v, out_ppl)

    cp_w.wait()
    cp_ri.wait()
    cp_rm.wait()
    acc = zero
    for k in range(MAXV // L):
        idx = ridx_v[pl.ds(k * L, L)]
        vals = plsc.load_gather(wll_full, [idx])
        mk = rmask_v[pl.ds(k * L, L)]
        acc = acc + vals * mk
    stage_v[...] = jnp.full((L,), jnp.sum(acc))
    pltpu.sync_copy(stage_v, out_ast.at[sid])


_ast_call = functools.partial(
    pl.kernel,
    out_type=[
        jax.ShapeDtypeStruct((NAST, L), jnp.float32),
        jax.ShapeDtypeStruct((L,), jnp.float32),
    ],
    mesh=plsc.VectorSubcoreMesh(core_axis_name="c", subcore_axis_name="s",
                                num_cores=1),
    compiler_params=pltpu.CompilerParams(needs_layout_passes=False),
    scratch_types=[
        pltpu.VMEM((TOTAL,), jnp.float32),
        pltpu.VMEM((MAXV,), jnp.int32),
        pltpu.VMEM((MAXV,), jnp.float32),
        pltpu.VMEM((NW, 4, L), jnp.float32),
        pltpu.VMEM((L,), jnp.float32),
        pltpu.SemaphoreType.DMA,
    ],
)(_ast_body)


def kernel(var_name_log_probs, variable_tgt_name_id, variable_tgt_name_weight,
           restoration_indices, restoration_mask):
    wll, m32 = _scan_call(var_name_log_probs, variable_tgt_name_id,
                          variable_tgt_name_weight)
    out_ast, out_ppl = _ast_call(wll, m32, restoration_indices,
                                 restoration_mask)
    return (out_ast[:, 0], out_ppl[0], out_ppl[1])


# speedup vs baseline: 2.5129x; 1.0453x over previous
import functools

import jax
import jax.numpy as jnp
from jax import lax
from jax.experimental import pallas as pl
from jax.experimental.pallas import tpu as pltpu
from jax.experimental.pallas import tpu_sc as plsc

TOTAL = 16384
VOCAB = 4096
NAST = 16
MAXV = 2048
NSUB = 16
NW = 32
L = 16
VARS_W = TOTAL // NW
CROWS = 8
NCHUNK = VARS_W // CROWS
NRING = 3


def _scan_body(tbl, ids, wts, out_wll, out_m,
               ids_v, w_v, wll_v, macc_v, bufs, sem_in, gsems):
    sid = lax.axis_index("s")
    cid = lax.axis_index("c")
    wid = sid * 2 + cid
    base = wid * VARS_W

    cp_ids = pltpu.async_copy(ids.at[pl.ds(base, VARS_W)], ids_v, sem_in)
    cp_w = pltpu.async_copy(wts.at[pl.ds(base, VARS_W)], w_v, sem_in)

    cps = [None] * NRING
    for b in range(NRING):
        cps[b] = pltpu.async_copy(tbl.at[pl.ds(base + b * CROWS, CROWS)],
                                  bufs.at[b], gsems.at[b])
    cp_ids.wait()
    cp_w.wait()

    iota = lax.iota(jnp.int32, L)
    low = iota < 8
    zero = jnp.zeros((L,), jnp.float32)
    one = jnp.ones((L,), jnp.float32)
    sr = zero
    nr = zero
    su = zero
    nu = zero
    pending = zero
    for c in range(NCHUNK):
        b = c % NRING
        p = c // 2
        cps[b].wait()
        v = ids_v[pl.ds(p * L, L)]
        vals = plsc.load_gather(bufs.at[b], [iota & 7, v])
        if c % 2 == 0:
            pending = vals
        else:
            ll = jnp.where(low, pending, vals)
            w = w_v[pl.ds(p * L, L)]
            rm = jnp.where(w == 1.0, one, zero)
            lr = ll * rm
            sr = sr + lr
            nr = nr + rm
            su = su + (ll - lr)
            nu = nu + (one - rm)
            wll_v[pl.ds(p * L, L)] = ll * w
        if c + NRING < NCHUNK:
            cps[b] = pltpu.async_copy(
                tbl.at[pl.ds(base + (c + NRING) * CROWS, CROWS)],
                bufs.at[b], gsems.at[b])
    macc_v[0, :] = sr
    macc_v[1, :] = nr
    macc_v[2, :] = su
    macc_v[3, :] = nu
    pltpu.sync_copy(wll_v, out_wll.at[pl.ds(base, VARS_W)])
    pltpu.sync_copy(macc_v, out_m.at[wid])


_scan_call = functools.partial(
    pl.kernel,
    out_type=[
        jax.ShapeDtypeStruct((TOTAL,), jnp.float32),
        jax.ShapeDtypeStruct((NW, 4, L), jnp.float32),
    ],
    mesh=plsc.VectorSubcoreMesh(core_axis_name="c", subcore_axis_name="s"),
    compiler_params=pltpu.CompilerParams(needs_layout_passes=False),
    scratch_types=[
        pltpu.VMEM((VARS_W,), jnp.int32),
        pltpu.VMEM((VARS_W,), jnp.float32),
        pltpu.VMEM((VARS_W,), jnp.float32),
        pltpu.VMEM((4, L), jnp.float32),
        pltpu.VMEM((NRING, CROWS, VOCAB), jnp.float32),
        pltpu.SemaphoreType.DMA,
        pltpu.SemaphoreType.DMA((NRING,)),
    ],
)(_scan_body)


def _ast_body(wll, m32, ridx, rmask, out_ast, out_ppl,
              wll_full, ridx_v, rmask_v, metrics_l, stage_v, sem):
    sid = lax.axis_index("s")
    iota = lax.iota(jnp.int32, L)
    zero = jnp.zeros((L,), jnp.float32)

    cp_w = pltpu.async_copy(wll, wll_full, sem)
    cp_ri = pltpu.async_copy(ridx.at[sid], ridx_v, sem)
    cp_rm = pltpu.async_copy(rmask.at[sid], rmask_v, sem)

    @pl.when(sid == 0)
    def _ppl():
        pltpu.sync_copy(m32, metrics_l)
        sr_t = zero
        nr_t = zero
        su_t = zero
        nu_t = zero
        for t in range(NW):
            sr_t = sr_t + metrics_l[t, 0, :]
            nr_t = nr_t + metrics_l[t, 1, :]
            su_t = su_t + metrics_l[t, 2, :]
            nu_t = nu_t + metrics_l[t, 3, :]
        ssr = jnp.full((L,), jnp.sum(sr_t))
        snr = jnp.full((L,), jnp.sum(nr_t))
        ssu = jnp.full((L,), jnp.sum(su_t))
        snu = jnp.full((L,), jnp.sum(nu_t))
        rv = jnp.exp(-(ssr / snr))
        uv = jnp.exp(-(ssu / snu))
        stage_v[...] = jnp.where(iota == 0, rv, jnp.where(iota == 1, uv, zero))
        pltpu.sync_copy(stage_v, out_ppl)

    cp_w.wait()
    cp_ri.wait()
    cp_rm.wait()
    acc = zero
    for k in range(MAXV // L):
        idx = ridx_v[pl.ds(k * L, L)]
        vals = plsc.load_gather(wll_full, [idx])
        mk = rmask_v[pl.ds(k * L, L)]
        acc = acc + vals * mk
    stage_v[...] = jnp.full((L,), jnp.sum(acc))
    pltpu.sync_copy(stage_v, out_ast.at[sid])


_ast_call = functools.partial(
    pl.kernel,
    out_type=[
        jax.ShapeDtypeStruct((NAST, L), jnp.float32),
        jax.ShapeDtypeStruct((L,), jnp.float32),
    ],
    mesh=plsc.VectorSubcoreMesh(core_axis_name="c", subcore_axis_name="s",
                                num_cores=1),
    compiler_params=pltpu.CompilerParams(needs_layout_passes=False),
    scratch_types=[
        pltpu.VMEM((TOTAL,), jnp.float32),
        pltpu.VMEM((MAXV,), jnp.int32),
        pltpu.VMEM((MAXV,), jnp.float32),
        pltpu.VMEM((NW, 4, L), jnp.float32),
        pltpu.VMEM((L,), jnp.float32),
        pltpu.SemaphoreType.DMA,
    ],
)(_ast_body)


def kernel(var_name_log_probs, variable_tgt_name_id, variable_tgt_name_weight,
           restoration_indices, restoration_mask):
    wll, m32 = _scan_call(var_name_log_probs, variable_tgt_name_id,
                          variable_tgt_name_weight)
    out_ast, out_ppl = _ast_call(wll, m32, restoration_indices,
                                 restoration_mask)
    return (out_ast[:, 0], out_ppl[0], out_ppl[1])
